# Initial kernel scaffold; baseline (speedup 1.0000x reference)
import jax
import jax.numpy as jnp
from jax.experimental import pallas as pl


def kernel(user, item, sentence, feature, feature_len, user_table, item_table, feature_table, sent_table, fc_w, fc_b):
    raise NotImplementedError("write your pallas kernel here")


# speedup vs baseline: 3.0928x; 3.0928x over previous
import functools

import jax
import jax.numpy as jnp
from jax import lax
from jax.experimental import pallas as pl
from jax.experimental.pallas import tpu as pltpu
from jax.experimental.pallas import tpu_sc as plsc

_NC = 2
_NS = 16
_NW = _NC * _NS
_L = 16


def _pf_body(ft_ref, wf_ref, pf_ref):
    pf_ref[...] = jnp.dot(ft_ref[...], wf_ref[...],
                          preferred_element_type=jnp.float32)


def _project_feature_table(ft_pad, w_f):
    vpad, d = ft_pad.shape
    out = pl.pallas_call(
        _pf_body,
        out_shape=jax.ShapeDtypeStruct((vpad, 1), jnp.float32),
    )(ft_pad, w_f.reshape(d, 1))
    return out.reshape(vpad)


def _make_sc_kernel(b, f_len, d_u, d_i, d_s, vpad):
    bpw = b // _NW
    n_grp = bpw // _L
    mesh = plsc.VectorSubcoreMesh(
        core_axis_name="c", subcore_axis_name="s",
        num_cores=_NC, num_subcores=_NS)

    @functools.partial(
        pl.kernel,
        out_type=jax.ShapeDtypeStruct((b,), jnp.float32),
        mesh=mesh,
        compiler_params=pltpu.CompilerParams(
            needs_layout_passes=False, use_tc_tiling_on_sc=False),
        scratch_types=[
            pltpu.VMEM((bpw,), jnp.int32),
            pltpu.VMEM((bpw,), jnp.int32),
            pltpu.VMEM((bpw,), jnp.int32),
            pltpu.VMEM((f_len, bpw), jnp.int32),
            pltpu.VMEM((bpw,), jnp.float32),
            pltpu.VMEM((bpw, d_u), jnp.float32),
            pltpu.VMEM((bpw, d_i), jnp.float32),
            pltpu.VMEM((bpw, d_s), jnp.float32),
            pltpu.VMEM((vpad,), jnp.float32),
            pltpu.VMEM((d_u + d_i + d_s + _L,), jnp.float32),
            pltpu.VMEM((bpw,), jnp.float32),
            pltpu.SemaphoreType.DMA,
            pltpu.SemaphoreType.DMA,
            pltpu.SemaphoreType.DMA,
        ],
    )
    def sc_kernel(uidx, iidx, sidx, fidx_t, flen, ut, it, st, pf, ws, out,
                  uidx_v, iidx_v, sidx_v, fidx_v, flen_v,
                  urows_v, irows_v, srows_v, pf_v, ws_v, out_v,
                  sem_u, sem_i, sem_s):
        wid = lax.axis_index("s") * _NC + lax.axis_index("c")
        base = wid * bpw

        pltpu.sync_copy(uidx.at[pl.ds(base, bpw)], uidx_v)
        pltpu.sync_copy(iidx.at[pl.ds(base, bpw)], iidx_v)
        pltpu.sync_copy(sidx.at[pl.ds(base, bpw)], sidx_v)
        cu = pltpu.async_copy(ut.at[uidx_v], urows_v, sem_u)
        ci = pltpu.async_copy(it.at[iidx_v], irows_v, sem_i)
        cs = pltpu.async_copy(st.at[sidx_v], srows_v, sem_s)

        pltpu.sync_copy(fidx_t.at[:, pl.ds(base, bpw)], fidx_v)
        pltpu.sync_copy(flen.at[pl.ds(base, bpw)], flen_v)
        pltpu.sync_copy(pf, pf_v)
        pltpu.sync_copy(ws, ws_v.at[pl.ds(0, d_u + d_i + d_s)])

        for g in range(n_grp):
            facc = jnp.zeros((_L,), jnp.float32)
            for f in range(f_len):
                idxv = fidx_v[f, pl.ds(g * _L, _L)]
                facc = facc + plsc.load_gather(pf_v, [idxv])
            out_v[pl.ds(g * _L, _L)] = facc / flen_v[pl.ds(g * _L, _L)]

        cu.wait()
        ci.wait()
        cs.wait()

        lanes = lax.iota(jnp.int32, _L)
        ridx = [g * _L + lanes for g in range(n_grp)]
        accs = tuple(out_v[pl.ds(g * _L, _L)] for g in range(n_grp))

        def mk_body(rows_v, woff):
            def body(d, accs):
                w = ws_v[pl.ds(woff + d, _L)][0]
                col = jnp.full((_L,), d, jnp.int32)
                return tuple(
                    a + plsc.load_gather(rows_v, [ridx[g], col]) * w
                    for g, a in enumerate(accs))
            return body

        accs = lax.fori_loop(0, d_u, mk_body(urows_v, 0), accs)
        accs = lax.fori_loop(0, d_i, mk_body(irows_v, d_u), accs)
        accs = lax.fori_loop(0, d_s, mk_body(srows_v, d_u + d_i), accs)
        for g in range(n_grp):
            out_v[pl.ds(g * _L, _L)] = accs[g]

        pltpu.sync_copy(out_v, out.at[pl.ds(base, bpw)])

    return sc_kernel


def kernel(user, item, sentence, feature, feature_len, user_table, item_table,
           feature_table, sent_table, fc_w, fc_b):
    b = user.shape[0]
    f_len = feature.shape[1]
    d_u = user_table.shape[1]
    d_i = item_table.shape[1]
    d_s = sent_table.shape[1]
    d_f = feature_table.shape[1]
    v_f = feature_table.shape[0]
    vpad = ((v_f + 1 + _L - 1) // _L) * _L

    w = fc_w.reshape(-1).astype(jnp.float32)
    w_u = w[:d_u]
    w_i = w[d_u:d_u + d_i]
    w_s = w[d_u + d_i:d_u + d_i + d_s]
    w_f = w[d_u + d_i + d_s:]
    ws = jnp.concatenate([w_u, w_i, w_s])

    ft_pad = jnp.pad(feature_table.astype(jnp.float32),
                     ((0, vpad - v_f), (0, 0)))
    pf = _project_feature_table(ft_pad, w_f)

    slot = jnp.arange(f_len, dtype=jnp.int32)[None, :]
    fmask = slot < feature_len.astype(jnp.int32)[:, None]
    fidx_t = jnp.where(fmask, feature.astype(jnp.int32), v_f).T
    flen_f = feature_len.astype(jnp.float32)

    sc = _make_sc_kernel(b, f_len, d_u, d_i, d_s, vpad)
    res = sc(user.astype(jnp.int32), item.astype(jnp.int32),
             sentence.astype(jnp.int32), fidx_t, flen_f,
             user_table.astype(jnp.float32), item_table.astype(jnp.float32),
             sent_table.astype(jnp.float32), pf, ws)
    return res.reshape(b, 1) + fc_b


# speedup vs baseline: 7.6390x; 2.4700x over previous
import functools

import jax
import jax.numpy as jnp
from jax import lax
from jax.experimental import pallas as pl
from jax.experimental.pallas import tpu as pltpu
from jax.experimental.pallas import tpu_sc as plsc

_NC = 2
_NS = 16
_NW = _NC * _NS
_L = 16

_BLK = 8192


def _proj_body(utt_ref, itt_ref, wu_ref, wi_ref, pu_ref, pi_ref):
    pu_ref[...] = jnp.sum(utt_ref[...] * wu_ref[...], axis=0)
    pi_ref[...] = jnp.sum(itt_ref[...] * wi_ref[...], axis=0)


def _project_ui(ut_t, it_t, w_u, w_i):
    d, v = ut_t.shape
    grid = (v + _BLK - 1) // _BLK
    return pl.pallas_call(
        _proj_body,
        grid=(grid,),
        in_specs=[
            pl.BlockSpec((d, _BLK), lambda j: (0, j)),
            pl.BlockSpec((d, _BLK), lambda j: (0, j)),
            pl.BlockSpec((d, 1), lambda j: (0, 0)),
            pl.BlockSpec((d, 1), lambda j: (0, 0)),
        ],
        out_specs=[
            pl.BlockSpec((_BLK,), lambda j: (j,)),
            pl.BlockSpec((_BLK,), lambda j: (j,)),
        ],
        out_shape=[
            jax.ShapeDtypeStruct((v,), jnp.float32),
            jax.ShapeDtypeStruct((v,), jnp.float32),
        ],
    )(ut_t, it_t, w_u.reshape(d, 1), w_i.reshape(d, 1))


def _pf_body(ftt_ref, wf_ref, pf_ref):
    pf_ref[...] = jnp.sum(ftt_ref[...] * wf_ref[...], axis=0)


def _project_feature_table(ft_t_pad, w_f):
    d, vpad = ft_t_pad.shape
    return pl.pallas_call(
        _pf_body,
        out_shape=jax.ShapeDtypeStruct((vpad,), jnp.float32),
    )(ft_t_pad, w_f.reshape(d, 1))


def _make_sc_kernel(b, f_len, d_s, vpad):
    bpw = b // _NW
    n_grp = bpw // _L
    mesh = plsc.VectorSubcoreMesh(
        core_axis_name="c", subcore_axis_name="s",
        num_cores=_NC, num_subcores=_NS)

    @functools.partial(
        pl.kernel,
        out_type=jax.ShapeDtypeStruct((b,), jnp.float32),
        mesh=mesh,
        compiler_params=pltpu.CompilerParams(
            needs_layout_passes=False, use_tc_tiling_on_sc=False),
        scratch_types=[
            pltpu.VMEM((bpw,), jnp.int32),
            pltpu.VMEM((bpw,), jnp.int32),
            pltpu.VMEM((bpw,), jnp.int32),
            pltpu.VMEM((f_len, bpw), jnp.int32),
            pltpu.VMEM((bpw,), jnp.float32),
            pltpu.VMEM((bpw,), jnp.float32),
            pltpu.VMEM((bpw,), jnp.float32),
            pltpu.VMEM((bpw, d_s), jnp.float32),
            pltpu.VMEM((vpad,), jnp.float32),
            pltpu.VMEM((d_s + _L,), jnp.float32),
            pltpu.VMEM((bpw,), jnp.float32),
            pltpu.SemaphoreType.DMA,
            pltpu.SemaphoreType.DMA,
            pltpu.SemaphoreType.DMA,
        ],
    )
    def sc_kernel(uidx, iidx, sidx, fidx_t, flen, pu, pi, st, pf, ws, out,
                  uidx_v, iidx_v, sidx_v, fidx_v, flen_v,
                  uval_v, ival_v, srows_v, pf_v, ws_v, out_v,
                  sem_u, sem_i, sem_s):
        wid = lax.axis_index("s") * _NC + lax.axis_index("c")
        base = wid * bpw

        pltpu.sync_copy(uidx.at[pl.ds(base, bpw)], uidx_v)
        pltpu.sync_copy(iidx.at[pl.ds(base, bpw)], iidx_v)
        pltpu.sync_copy(sidx.at[pl.ds(base, bpw)], sidx_v)
        cu = pltpu.async_copy(pu.at[uidx_v], uval_v, sem_u)
        ci = pltpu.async_copy(pi.at[iidx_v], ival_v, sem_i)
        cs = pltpu.async_copy(st.at[sidx_v], srows_v, sem_s)

        pltpu.sync_copy(fidx_t.at[:, pl.ds(base, bpw)], fidx_v)
        pltpu.sync_copy(flen.at[pl.ds(base, bpw)], flen_v)
        pltpu.sync_copy(pf, pf_v)
        pltpu.sync_copy(ws, ws_v.at[pl.ds(0, d_s)])

        for g in range(n_grp):
            facc = jnp.zeros((_L,), jnp.float32)
            for f in range(f_len):
                idxv = fidx_v[f, pl.ds(g * _L, _L)]
                facc = facc + plsc.load_gather(pf_v, [idxv])
            out_v[pl.ds(g * _L, _L)] = facc / flen_v[pl.ds(g * _L, _L)]

        cu.wait()
        ci.wait()
        cs.wait()

        lanes = lax.iota(jnp.int32, _L)
        ridx = [g * _L + lanes for g in range(n_grp)]
        accs = tuple(
            out_v[pl.ds(g * _L, _L)]
            + uval_v[pl.ds(g * _L, _L)] + ival_v[pl.ds(g * _L, _L)]
            for g in range(n_grp))

        def body(d, accs):
            w = ws_v[pl.ds(d, _L)][0]
            col = jnp.full((_L,), d, jnp.int32)
            return tuple(
                a + plsc.load_gather(srows_v, [ridx[g], col]) * w
                for g, a in enumerate(accs))

        accs = lax.fori_loop(0, d_s, body, accs)
        for g in range(n_grp):
            out_v[pl.ds(g * _L, _L)] = accs[g]

        pltpu.sync_copy(out_v, out.at[pl.ds(base, bpw)])

    return sc_kernel


def kernel(user, item, sentence, feature, feature_len, user_table, item_table,
           feature_table, sent_table, fc_w, fc_b):
    b = user.shape[0]
    f_len = feature.shape[1]
    d_u = user_table.shape[1]
    d_i = item_table.shape[1]
    d_s = sent_table.shape[1]
    v_f = feature_table.shape[0]
    vpad = ((v_f + 1 + _L - 1) // _L) * _L

    w = fc_w.reshape(-1).astype(jnp.float32)
    w_u = w[:d_u]
    w_i = w[d_u:d_u + d_i]
    w_s = w[d_u + d_i:d_u + d_i + d_s]
    w_f = w[d_u + d_i + d_s:]

    pu, pi = _project_ui(user_table.T, item_table.T, w_u, w_i)
    ft_t_pad = jnp.pad(feature_table.T.astype(jnp.float32),
                       ((0, 0), (0, vpad - v_f)))
    pf = _project_feature_table(ft_t_pad, w_f)

    slot = jnp.arange(f_len, dtype=jnp.int32)[None, :]
    fmask = slot < feature_len.astype(jnp.int32)[:, None]
    fidx_t = jnp.where(fmask, feature.astype(jnp.int32), v_f).T
    flen_f = feature_len.astype(jnp.float32)

    sc = _make_sc_kernel(b, f_len, d_s, vpad)
    res = sc(user.astype(jnp.int32), item.astype(jnp.int32),
             sentence.astype(jnp.int32), fidx_t, flen_f,
             pu, pi, sent_table.astype(jnp.float32), pf, w_s)
    return res.reshape(b, 1) + fc_b


# speedup vs baseline: 7.7664x; 1.0167x over previous
import functools

import jax
import jax.numpy as jnp
from jax import lax
from jax.experimental import pallas as pl
from jax.experimental.pallas import tpu as pltpu
from jax.experimental.pallas import tpu_sc as plsc

_NC = 2
_NS = 16
_NW = _NC * _NS
_L = 16

_BLK = 8192


def _proj_body(utt_ref, itt_ref, wu_ref, wi_ref, pu_ref, pi_ref):
    pu_ref[...] = jnp.sum(utt_ref[...] * wu_ref[...], axis=0)
    pi_ref[...] = jnp.sum(itt_ref[...] * wi_ref[...], axis=0)


def _project_ui(ut_t, it_t, w_u, w_i):
    d, v = ut_t.shape
    grid = (v + _BLK - 1) // _BLK
    return pl.pallas_call(
        _proj_body,
        grid=(grid,),
        in_specs=[
            pl.BlockSpec((d, _BLK), lambda j: (0, j)),
            pl.BlockSpec((d, _BLK), lambda j: (0, j)),
            pl.BlockSpec((d, 1), lambda j: (0, 0)),
            pl.BlockSpec((d, 1), lambda j: (0, 0)),
        ],
        out_specs=[
            pl.BlockSpec((_BLK,), lambda j: (j,)),
            pl.BlockSpec((_BLK,), lambda j: (j,)),
        ],
        out_shape=[
            jax.ShapeDtypeStruct((v,), jnp.float32),
            jax.ShapeDtypeStruct((v,), jnp.float32),
        ],
    )(ut_t, it_t, w_u.reshape(d, 1), w_i.reshape(d, 1))


def _pf_body(ftt_ref, wf_ref, pf_ref):
    pf_ref[...] = jnp.sum(ftt_ref[...] * wf_ref[...], axis=0)


def _project_feature_table(ft_t_pad, w_f):
    d, vpad = ft_t_pad.shape
    return pl.pallas_call(
        _pf_body,
        out_shape=jax.ShapeDtypeStruct((vpad,), jnp.float32),
    )(ft_t_pad, w_f.reshape(d, 1))


def _make_sc_kernel(b, f_len, d_s, vpad):
    bpw = b // _NW
    n_grp = bpw // _L
    mesh = plsc.VectorSubcoreMesh(
        core_axis_name="c", subcore_axis_name="s",
        num_cores=_NC, num_subcores=_NS)

    @functools.partial(
        pl.kernel,
        out_type=jax.ShapeDtypeStruct((b,), jnp.float32),
        mesh=mesh,
        compiler_params=pltpu.CompilerParams(
            needs_layout_passes=False, use_tc_tiling_on_sc=False),
        scratch_types=[
            pltpu.VMEM((bpw,), jnp.int32),
            pltpu.VMEM((bpw,), jnp.int32),
            pltpu.VMEM((bpw,), jnp.int32),
            pltpu.VMEM((f_len, bpw), jnp.int32),
            pltpu.VMEM((bpw,), jnp.float32),
            pltpu.VMEM((bpw,), jnp.float32),
            pltpu.VMEM((bpw,), jnp.float32),
            pltpu.VMEM((bpw, d_s), jnp.float32),
            pltpu.VMEM((vpad,), jnp.float32),
            pltpu.VMEM((d_s + _L,), jnp.float32),
            pltpu.VMEM((bpw,), jnp.float32),
            pltpu.SemaphoreType.DMA,
            pltpu.SemaphoreType.DMA,
            pltpu.SemaphoreType.DMA,
        ],
    )
    def sc_kernel(uidx, iidx, sidx, fidx_t, flen, pu, pi, st, pf, ws, out,
                  uidx_v, iidx_v, sidx_v, fidx_v, flen_v,
                  uval_v, ival_v, srows_v, pf_v, ws_v, out_v,
                  sem_u, sem_i, sem_s):
        wid = lax.axis_index("s") * _NC + lax.axis_index("c")
        base = wid * bpw

        pltpu.sync_copy(uidx.at[pl.ds(base, bpw)], uidx_v)
        pltpu.sync_copy(iidx.at[pl.ds(base, bpw)], iidx_v)
        pltpu.sync_copy(sidx.at[pl.ds(base, bpw)], sidx_v)
        cu = pltpu.async_copy(pu.at[uidx_v], uval_v, sem_u)
        ci = pltpu.async_copy(pi.at[iidx_v], ival_v, sem_i)
        cs = pltpu.async_copy(st.at[sidx_v], srows_v, sem_s)

        pltpu.sync_copy(fidx_t.at[:, pl.ds(base, bpw)], fidx_v)
        pltpu.sync_copy(flen.at[pl.ds(base, bpw)], flen_v)
        pltpu.sync_copy(pf, pf_v)
        pltpu.sync_copy(ws, ws_v.at[pl.ds(0, d_s)])

        for g in range(n_grp):
            facc = jnp.zeros((_L,), jnp.float32)
            lv = flen_v[pl.ds(g * _L, _L)]
            for f in range(f_len):
                idxv = fidx_v[f, pl.ds(g * _L, _L)]
                vals = plsc.load_gather(pf_v, [idxv])
                facc = facc + jnp.where(jnp.float32(f) < lv, vals, 0.0)
            out_v[pl.ds(g * _L, _L)] = facc / lv

        cu.wait()
        ci.wait()
        cs.wait()

        lanes = lax.iota(jnp.int32, _L)
        ridx = [g * _L + lanes for g in range(n_grp)]
        accs = tuple(
            out_v[pl.ds(g * _L, _L)]
            + uval_v[pl.ds(g * _L, _L)] + ival_v[pl.ds(g * _L, _L)]
            for g in range(n_grp))

        def body(_, carry):
            col, *accs = carry
            wv = plsc.load_gather(ws_v, [col])
            accs = [
                a + plsc.load_gather(srows_v, [ridx[g], col]) * wv
                for g, a in enumerate(accs)]
            return (col + 1, *accs)

        carry = lax.fori_loop(
            0, d_s, body, (jnp.zeros((_L,), jnp.int32), *accs), unroll=4)
        for g in range(n_grp):
            out_v[pl.ds(g * _L, _L)] = carry[1 + g]

        pltpu.sync_copy(out_v, out.at[pl.ds(base, bpw)])

    return sc_kernel


def kernel(user, item, sentence, feature, feature_len, user_table, item_table,
           feature_table, sent_table, fc_w, fc_b):
    b = user.shape[0]
    f_len = feature.shape[1]
    d_u = user_table.shape[1]
    d_i = item_table.shape[1]
    d_s = sent_table.shape[1]
    v_f = feature_table.shape[0]
    vpad = ((v_f + 1 + _L - 1) // _L) * _L

    w = fc_w.reshape(-1).astype(jnp.float32)
    w_u = w[:d_u]
    w_i = w[d_u:d_u + d_i]
    w_s = w[d_u + d_i:d_u + d_i + d_s]
    w_f = w[d_u + d_i + d_s:]

    pu, pi = _project_ui(user_table.T, item_table.T, w_u, w_i)
    ft_t_pad = jnp.pad(feature_table.T.astype(jnp.float32),
                       ((0, 0), (0, vpad - v_f)))
    pf = _project_feature_table(ft_t_pad, w_f)

    fidx_t = feature.astype(jnp.int32).T
    flen_f = feature_len.astype(jnp.float32)

    sc = _make_sc_kernel(b, f_len, d_s, vpad)
    res = sc(user.astype(jnp.int32), item.astype(jnp.int32),
             sentence.astype(jnp.int32), fidx_t, flen_f,
             pu, pi, sent_table.astype(jnp.float32), pf, w_s)
    return res.reshape(b, 1) + fc_b


# speedup vs baseline: 8.0451x; 1.0359x over previous
import functools

import jax
import jax.numpy as jnp
from jax import lax
from jax.experimental import pallas as pl
from jax.experimental.pallas import tpu as pltpu
from jax.experimental.pallas import tpu_sc as plsc

_NC = 2
_NS = 16
_NW = _NC * _NS
_L = 16

_BLK = 8192


def _proj_body(utt_ref, itt_ref, wu_ref, wi_ref, pu_ref, pi_ref):
    pu_ref[...] = jnp.sum(utt_ref[...] * wu_ref[...], axis=0)
    pi_ref[...] = jnp.sum(itt_ref[...] * wi_ref[...], axis=0)


def _project_ui(ut_t, it_t, w_u, w_i):
    d, v = ut_t.shape
    grid = (v + _BLK - 1) // _BLK
    return pl.pallas_call(
        _proj_body,
        grid=(grid,),
        in_specs=[
            pl.BlockSpec((d, _BLK), lambda j: (0, j)),
            pl.BlockSpec((d, _BLK), lambda j: (0, j)),
            pl.BlockSpec((d, 1), lambda j: (0, 0)),
            pl.BlockSpec((d, 1), lambda j: (0, 0)),
        ],
        out_specs=[
            pl.BlockSpec((_BLK,), lambda j: (j,)),
            pl.BlockSpec((_BLK,), lambda j: (j,)),
        ],
        out_shape=[
            jax.ShapeDtypeStruct((v,), jnp.float32),
            jax.ShapeDtypeStruct((v,), jnp.float32),
        ],
    )(ut_t, it_t, w_u.reshape(d, 1), w_i.reshape(d, 1))


def _pf_body(ftt_ref, wf_ref, pf_ref):
    pf_ref[...] = jnp.sum(ftt_ref[...] * wf_ref[...], axis=0)


def _project_feature_table(ft_t_pad, w_f):
    d, vpad = ft_t_pad.shape
    return pl.pallas_call(
        _pf_body,
        out_shape=jax.ShapeDtypeStruct((vpad,), jnp.float32),
    )(ft_t_pad, w_f.reshape(d, 1))


def _make_sc_kernel(b, f_len, d_s, vpad):
    bpw = b // _NW
    n_grp = bpw // _L
    mesh = plsc.VectorSubcoreMesh(
        core_axis_name="c", subcore_axis_name="s",
        num_cores=_NC, num_subcores=_NS)

    @functools.partial(
        pl.kernel,
        out_type=jax.ShapeDtypeStruct((b,), jnp.float32),
        mesh=mesh,
        compiler_params=pltpu.CompilerParams(
            needs_layout_passes=False, use_tc_tiling_on_sc=False),
        scratch_types=[
            pltpu.VMEM((bpw,), jnp.int32),
            pltpu.VMEM((bpw,), jnp.int32),
            pltpu.VMEM((bpw,), jnp.int32),
            pltpu.VMEM((f_len, bpw), jnp.int32),
            pltpu.VMEM((bpw,), jnp.float32),
            pltpu.VMEM((bpw,), jnp.float32),
            pltpu.VMEM((bpw,), jnp.float32),
            pltpu.VMEM((bpw, d_s), jnp.float32),
            pltpu.VMEM((vpad,), jnp.float32),
            pltpu.VMEM((d_s + _L,), jnp.float32),
            pltpu.VMEM((bpw,), jnp.float32),
            [pltpu.SemaphoreType.DMA] * 7,
            [pltpu.SemaphoreType.DMA] * 2,
            [pltpu.SemaphoreType.DMA] * 4,
        ],
    )
    def sc_kernel(uidx, iidx, sidx, fidx_t, flen, pu, pi, st, pf, ws, out,
                  uidx_v, iidx_v, sidx_v, fidx_v, flen_v,
                  uval_v, ival_v, srows_v, pf_v, ws_v, out_v,
                  sems, sems_ui, sems_s):
        wid = lax.axis_index("s") * _NC + lax.axis_index("c")
        base = wid * bpw

        c_stage = [
            pltpu.async_copy(uidx.at[pl.ds(base, bpw)], uidx_v, sems[0]),
            pltpu.async_copy(iidx.at[pl.ds(base, bpw)], iidx_v, sems[1]),
            pltpu.async_copy(sidx.at[pl.ds(base, bpw)], sidx_v, sems[2]),
            pltpu.async_copy(fidx_t.at[:, pl.ds(base, bpw)], fidx_v, sems[3]),
            pltpu.async_copy(flen.at[pl.ds(base, bpw)], flen_v, sems[4]),
            pltpu.async_copy(pf, pf_v, sems[5]),
            pltpu.async_copy(ws, ws_v.at[pl.ds(0, d_s)], sems[6]),
        ]
        c_stage[0].wait()
        cu = pltpu.async_copy(pu.at[uidx_v], uval_v, sems_ui[0])
        c_stage[1].wait()
        ci = pltpu.async_copy(pi.at[iidx_v], ival_v, sems_ui[1])
        c_stage[2].wait()
        n_sp = len(sems_s)
        spr = bpw // n_sp
        cs = [
            pltpu.async_copy(st.at[sidx_v.at[pl.ds(k * spr, spr)]],
                             srows_v.at[pl.ds(k * spr, spr)], sems_s[k])
            for k in range(n_sp)]
        for c in c_stage[3:]:
            c.wait()

        for g in range(n_grp):
            facc = jnp.zeros((_L,), jnp.float32)
            lv = flen_v[pl.ds(g * _L, _L)]
            for f in range(f_len):
                idxv = fidx_v[f, pl.ds(g * _L, _L)]
                vals = plsc.load_gather(pf_v, [idxv])
                facc = facc + jnp.where(jnp.float32(f) < lv, vals, 0.0)
            out_v[pl.ds(g * _L, _L)] = facc / lv

        cu.wait()
        ci.wait()
        for c in cs:
            c.wait()

        lanes = lax.iota(jnp.int32, _L)
        ridx = [g * _L + lanes for g in range(n_grp)]
        accs = tuple(
            out_v[pl.ds(g * _L, _L)]
            + uval_v[pl.ds(g * _L, _L)] + ival_v[pl.ds(g * _L, _L)]
            for g in range(n_grp))

        def body(_, carry):
            col, *accs = carry
            wv = plsc.load_gather(ws_v, [col])
            accs = [
                a + plsc.load_gather(srows_v, [ridx[g], col]) * wv
                for g, a in enumerate(accs)]
            return (col + 1, *accs)

        carry = lax.fori_loop(
            0, d_s, body, (jnp.zeros((_L,), jnp.int32), *accs), unroll=4)
        for g in range(n_grp):
            out_v[pl.ds(g * _L, _L)] = carry[1 + g]

        pltpu.sync_copy(out_v, out.at[pl.ds(base, bpw)])

    return sc_kernel


def kernel(user, item, sentence, feature, feature_len, user_table, item_table,
           feature_table, sent_table, fc_w, fc_b):
    b = user.shape[0]
    f_len = feature.shape[1]
    d_u = user_table.shape[1]
    d_i = item_table.shape[1]
    d_s = sent_table.shape[1]
    v_f = feature_table.shape[0]
    vpad = ((v_f + 1 + _L - 1) // _L) * _L

    w = fc_w.reshape(-1).astype(jnp.float32)
    w_u = w[:d_u]
    w_i = w[d_u:d_u + d_i]
    w_s = w[d_u + d_i:d_u + d_i + d_s]
    w_f = w[d_u + d_i + d_s:]

    pu, pi = _project_ui(user_table.T, item_table.T, w_u, w_i)
    ft_t_pad = jnp.pad(feature_table.T.astype(jnp.float32),
                       ((0, 0), (0, vpad - v_f)))
    pf = _project_feature_table(ft_t_pad, w_f)

    fidx_t = feature.astype(jnp.int32).T
    flen_f = feature_len.astype(jnp.float32)

    sc = _make_sc_kernel(b, f_len, d_s, vpad)
    res = sc(user.astype(jnp.int32), item.astype(jnp.int32),
             sentence.astype(jnp.int32), fidx_t, flen_f,
             pu, pi, sent_table.astype(jnp.float32), pf, w_s)
    return res.reshape(b, 1) + fc_b


# speedup vs baseline: 8.5816x; 1.0667x over previous
import functools

import jax
import jax.numpy as jnp
from jax import lax
from jax.experimental import pallas as pl
from jax.experimental.pallas import tpu as pltpu
from jax.experimental.pallas import tpu_sc as plsc

_NC = 2
_NS = 16
_NW = _NC * _NS
_L = 16

_BLK = 16384


def _proj_body(utt_ref, itt_ref, ftt_ref, wu_ref, wi_ref, wf_ref,
               pu_ref, pi_ref, pf_ref):
    pu_ref[...] = jnp.dot(wu_ref[...], utt_ref[...],
                          preferred_element_type=jnp.float32)
    pi_ref[...] = jnp.dot(wi_ref[...], itt_ref[...],
                          preferred_element_type=jnp.float32)

    @pl.when(pl.program_id(0) == 0)
    def _():
        pf_ref[...] = jnp.dot(wf_ref[...], ftt_ref[...],
                              preferred_element_type=jnp.float32)


def _project_tables(ut_t, it_t, ft_t, w_u, w_i, w_f):
    d, v = ut_t.shape
    v_f = ft_t.shape[1]
    grid = (v + _BLK - 1) // _BLK
    pu, pi, pf = pl.pallas_call(
        _proj_body,
        grid=(grid,),
        in_specs=[
            pl.BlockSpec((d, _BLK), lambda j: (0, j)),
            pl.BlockSpec((d, _BLK), lambda j: (0, j)),
            pl.BlockSpec((d, v_f), lambda j: (0, 0)),
            pl.BlockSpec((1, d), lambda j: (0, 0)),
            pl.BlockSpec((1, d), lambda j: (0, 0)),
            pl.BlockSpec((1, d), lambda j: (0, 0)),
        ],
        out_specs=[
            pl.BlockSpec((1, _BLK), lambda j: (0, j)),
            pl.BlockSpec((1, _BLK), lambda j: (0, j)),
            pl.BlockSpec((1, v_f), lambda j: (0, 0)),
        ],
        out_shape=[
            jax.ShapeDtypeStruct((1, v), jnp.float32),
            jax.ShapeDtypeStruct((1, v), jnp.float32),
            jax.ShapeDtypeStruct((1, v_f), jnp.float32),
        ],
    )(ut_t, it_t, ft_t, w_u, w_i, w_f)
    return pu.reshape(v), pi.reshape(v), pf.reshape(v_f)


def _mesh():
    return plsc.VectorSubcoreMesh(
        core_axis_name="c", subcore_axis_name="s",
        num_cores=_NC, num_subcores=_NS)


def _make_sc1(b, f_len, d_s, v_f):
    bpw = b // _NW
    n_grp = bpw // _L

    @functools.partial(
        pl.kernel,
        out_type=jax.ShapeDtypeStruct((b,), jnp.float32),
        mesh=_mesh(),
        compiler_params=pltpu.CompilerParams(
            needs_layout_passes=False, use_tc_tiling_on_sc=False),
        scratch_types=[
            pltpu.VMEM((bpw,), jnp.int32),
            pltpu.VMEM((f_len, bpw), jnp.int32),
            pltpu.VMEM((bpw,), jnp.int32),
            pltpu.VMEM((bpw, d_s), jnp.float32),
            pltpu.VMEM((v_f,), jnp.float32),
            pltpu.VMEM((d_s + _L,), jnp.float32),
            pltpu.VMEM((bpw,), jnp.float32),
            [pltpu.SemaphoreType.DMA] * 5,
            [pltpu.SemaphoreType.DMA] * 4,
        ],
    )
    def sc1(sidx, fidx_t, flen, st, pf, ws, out,
            sidx_v, fidx_v, flen_v, srows_v, pf_v, ws_v, out_v,
            sems, sems_s):
        wid = lax.axis_index("s") * _NC + lax.axis_index("c")
        base = wid * bpw

        c_stage = [
            pltpu.async_copy(sidx.at[pl.ds(base, bpw)], sidx_v, sems[0]),
            pltpu.async_copy(fidx_t.at[:, pl.ds(base, bpw)], fidx_v, sems[1]),
            pltpu.async_copy(flen.at[pl.ds(base, bpw)], flen_v, sems[2]),
            pltpu.async_copy(pf, pf_v, sems[3]),
            pltpu.async_copy(ws, ws_v.at[pl.ds(0, d_s)], sems[4]),
        ]
        c_stage[0].wait()
        n_sp = len(sems_s)
        spr = bpw // n_sp
        cs = [
            pltpu.async_copy(st.at[sidx_v.at[pl.ds(k * spr, spr)]],
                             srows_v.at[pl.ds(k * spr, spr)], sems_s[k])
            for k in range(n_sp)]
        for c in c_stage[1:]:
            c.wait()

        for g in range(n_grp):
            facc = jnp.zeros((_L,), jnp.float32)
            lv = flen_v[pl.ds(g * _L, _L)].astype(jnp.float32)
            for f in range(f_len):
                idxv = fidx_v[f, pl.ds(g * _L, _L)]
                vals = plsc.load_gather(pf_v, [idxv])
                facc = facc + jnp.where(jnp.float32(f) < lv, vals, 0.0)
            out_v[pl.ds(g * _L, _L)] = facc / lv

        for c in cs:
            c.wait()

        lanes = lax.iota(jnp.int32, _L)
        ridx = [g * _L + lanes for g in range(n_grp)]
        accs = tuple(out_v[pl.ds(g * _L, _L)] for g in range(n_grp))

        def body(_, carry):
            col, *accs = carry
            wv = plsc.load_gather(ws_v, [col])
            accs = [
                a + plsc.load_gather(srows_v, [ridx[g], col]) * wv
                for g, a in enumerate(accs)]
            return (col + 1, *accs)

        carry = lax.fori_loop(
            0, d_s, body, (jnp.zeros((_L,), jnp.int32), *accs), unroll=4)
        for g in range(n_grp):
            out_v[pl.ds(g * _L, _L)] = carry[1 + g]

        pltpu.sync_copy(out_v, out.at[pl.ds(base, bpw)])

    return sc1


def _make_sc2(b):
    bpw = b // _NW
    n_grp = bpw // _L

    @functools.partial(
        pl.kernel,
        out_type=jax.ShapeDtypeStruct((b,), jnp.float32),
        mesh=_mesh(),
        compiler_params=pltpu.CompilerParams(
            needs_layout_passes=False, use_tc_tiling_on_sc=False),
        scratch_types=[
            pltpu.VMEM((bpw,), jnp.int32),
            pltpu.VMEM((bpw,), jnp.int32),
            pltpu.VMEM((bpw,), jnp.float32),
            pltpu.VMEM((bpw,), jnp.float32),
            pltpu.VMEM((bpw,), jnp.float32),
            [pltpu.SemaphoreType.DMA] * 5,
        ],
    )
    def sc2(uidx, iidx, part, pu, pi, out,
            uidx_v, iidx_v, part_v, uval_v, ival_v, sems):
        wid = lax.axis_index("s") * _NC + lax.axis_index("c")
        base = wid * bpw

        cu0 = pltpu.async_copy(uidx.at[pl.ds(base, bpw)], uidx_v, sems[0])
        ci0 = pltpu.async_copy(iidx.at[pl.ds(base, bpw)], iidx_v, sems[1])
        cp = pltpu.async_copy(part.at[pl.ds(base, bpw)], part_v, sems[2])
        cu0.wait()
        cu = pltpu.async_copy(pu.at[uidx_v], uval_v, sems[3])
        ci0.wait()
        ci = pltpu.async_copy(pi.at[iidx_v], ival_v, sems[4])
        cp.wait()
        cu.wait()
        ci.wait()
        for g in range(n_grp):
            sl = pl.ds(g * _L, _L)
            part_v[sl] = part_v[sl] + uval_v[sl] + ival_v[sl]
        pltpu.sync_copy(part_v, out.at[pl.ds(base, bpw)])

    return sc2


def kernel(user, item, sentence, feature, feature_len, user_table, item_table,
           feature_table, sent_table, fc_w, fc_b):
    b = user.shape[0]
    f_len = feature.shape[1]
    d_u = user_table.shape[1]
    d_i = item_table.shape[1]
    d_s = sent_table.shape[1]
    v_f = feature_table.shape[0]

    fcw = fc_w.astype(jnp.float32)
    w_u = fcw[:, :d_u]
    w_i = fcw[:, d_u:d_u + d_i]
    w_s = fcw[0, d_u + d_i:d_u + d_i + d_s]
    w_f = fcw[:, d_u + d_i + d_s:]

    pu, pi, pf = _project_tables(
        user_table.T, item_table.T, feature_table.T, w_u, w_i, w_f)

    fidx_t = feature.astype(jnp.int32).T
    flen_i = feature_len.astype(jnp.int32)

    sc1 = _make_sc1(b, f_len, d_s, v_f)
    part = sc1(sentence.astype(jnp.int32), fidx_t, flen_i,
               sent_table.astype(jnp.float32), pf, w_s)
    sc2 = _make_sc2(b)
    res = sc2(user.astype(jnp.int32), item.astype(jnp.int32), part, pu, pi)
    return res.reshape(b, 1) + fc_b


# speedup vs baseline: 9.9945x; 1.1646x over previous
import functools

import jax
import jax.numpy as jnp
from jax import lax
from jax.experimental import pallas as pl
from jax.experimental.pallas import tpu as pltpu
from jax.experimental.pallas import tpu_sc as plsc

_NC = 2
_NS = 16
_NW = _NC * _NS
_L = 16

_BLK = 16384


def _proj_body(utt_ref, itt_ref, wu_ref, wi_ref, pu_ref, pi_ref):
    pu_ref[...] = jnp.dot(wu_ref[...], utt_ref[...],
                          preferred_element_type=jnp.float32)
    pi_ref[...] = jnp.dot(wi_ref[...], itt_ref[...],
                          preferred_element_type=jnp.float32)


def _project_tables(ut_t, it_t, w_u, w_i):
    d, v = ut_t.shape
    grid = (v + _BLK - 1) // _BLK
    pu, pi = pl.pallas_call(
        _proj_body,
        grid=(grid,),
        in_specs=[
            pl.BlockSpec((d, _BLK), lambda j: (0, j)),
            pl.BlockSpec((d, _BLK), lambda j: (0, j)),
            pl.BlockSpec((1, d), lambda j: (0, 0)),
            pl.BlockSpec((1, d), lambda j: (0, 0)),
        ],
        out_specs=[
            pl.BlockSpec((1, _BLK), lambda j: (0, j)),
            pl.BlockSpec((1, _BLK), lambda j: (0, j)),
        ],
        out_shape=[
            jax.ShapeDtypeStruct((1, v), jnp.float32),
            jax.ShapeDtypeStruct((1, v), jnp.float32),
        ],
    )(ut_t, it_t, w_u, w_i)
    return pu.reshape(v), pi.reshape(v)


def _pf_body(ftt_ref, wf_ref, pf_ref):
    pf_ref[...] = jnp.dot(wf_ref[...], ftt_ref[...],
                          preferred_element_type=jnp.float32)


def _project_feature_table(ft_t, w_f):
    d, v_f = ft_t.shape
    pf = pl.pallas_call(
        _pf_body,
        out_shape=jax.ShapeDtypeStruct((1, v_f), jnp.float32),
    )(ft_t, w_f)
    return pf.reshape(v_f)


def _mesh():
    return plsc.VectorSubcoreMesh(
        core_axis_name="c", subcore_axis_name="s",
        num_cores=_NC, num_subcores=_NS)


def _make_sc1(b, f_len, d_s, v_f):
    bpw = b // _NW
    n_grp = bpw // _L

    @functools.partial(
        pl.kernel,
        out_type=jax.ShapeDtypeStruct((b,), jnp.float32),
        mesh=_mesh(),
        compiler_params=pltpu.CompilerParams(
            needs_layout_passes=False, use_tc_tiling_on_sc=False),
        scratch_types=[
            pltpu.VMEM((bpw,), jnp.int32),
            pltpu.VMEM((f_len, bpw), jnp.int32),
            pltpu.VMEM((bpw,), jnp.int32),
            pltpu.VMEM((bpw, d_s), jnp.float32),
            pltpu.VMEM((v_f,), jnp.float32),
            pltpu.VMEM((d_s + _L,), jnp.float32),
            pltpu.VMEM((bpw,), jnp.float32),
            [pltpu.SemaphoreType.DMA] * 5,
            [pltpu.SemaphoreType.DMA] * 4,
        ],
    )
    def sc1(sidx, fidx_t, flen, st, pf, ws, out,
            sidx_v, fidx_v, flen_v, srows_v, pf_v, ws_v, out_v,
            sems, sems_s):
        wid = lax.axis_index("s") * _NC + lax.axis_index("c")
        base = wid * bpw

        c_stage = [
            pltpu.async_copy(sidx.at[pl.ds(base, bpw)], sidx_v, sems[0]),
            pltpu.async_copy(fidx_t.at[:, pl.ds(base, bpw)], fidx_v, sems[1]),
            pltpu.async_copy(flen.at[pl.ds(base, bpw)], flen_v, sems[2]),
            pltpu.async_copy(pf, pf_v, sems[3]),
            pltpu.async_copy(ws, ws_v.at[pl.ds(0, d_s)], sems[4]),
        ]
        c_stage[0].wait()
        n_sp = len(sems_s)
        spr = bpw // n_sp
        cs = [
            pltpu.async_copy(st.at[sidx_v.at[pl.ds(k * spr, spr)]],
                             srows_v.at[pl.ds(k * spr, spr)], sems_s[k])
            for k in range(n_sp)]
        for c in c_stage[1:]:
            c.wait()

        for g in range(n_grp):
            facc = jnp.zeros((_L,), jnp.float32)
            lv = flen_v[pl.ds(g * _L, _L)].astype(jnp.float32)
            for f in range(f_len):
                idxv = fidx_v[f, pl.ds(g * _L, _L)]
                vals = plsc.load_gather(pf_v, [idxv])
                facc = facc + jnp.where(jnp.float32(f) < lv, vals, 0.0)
            out_v[pl.ds(g * _L, _L)] = facc / lv

        for c in cs:
            c.wait()

        lanes = lax.iota(jnp.int32, _L)
        ridx = [g * _L + lanes for g in range(n_grp)]
        accs = tuple(out_v[pl.ds(g * _L, _L)] for g in range(n_grp))

        def body(_, carry):
            col, *accs = carry
            wv = plsc.load_gather(ws_v, [col])
            accs = [
                a + plsc.load_gather(srows_v, [ridx[g], col]) * wv
                for g, a in enumerate(accs)]
            return (col + 1, *accs)

        carry = lax.fori_loop(
            0, d_s, body, (jnp.zeros((_L,), jnp.int32), *accs), unroll=4)
        for g in range(n_grp):
            out_v[pl.ds(g * _L, _L)] = carry[1 + g]

        pltpu.sync_copy(out_v, out.at[pl.ds(base, bpw)])

    return sc1


def _make_sc2(b):
    bpw = b // _NW
    n_grp = bpw // _L

    @functools.partial(
        pl.kernel,
        out_type=jax.ShapeDtypeStruct((b,), jnp.float32),
        mesh=_mesh(),
        compiler_params=pltpu.CompilerParams(
            needs_layout_passes=False, use_tc_tiling_on_sc=False),
        scratch_types=[
            pltpu.VMEM((bpw,), jnp.int32),
            pltpu.VMEM((bpw,), jnp.int32),
            pltpu.VMEM((bpw,), jnp.float32),
            pltpu.VMEM((bpw,), jnp.float32),
            pltpu.VMEM((bpw,), jnp.float32),
            pltpu.VMEM((_L,), jnp.float32),
            [pltpu.SemaphoreType.DMA] * 6,
        ],
    )
    def sc2(uidx, iidx, part, pu, pi, bias, out,
            uidx_v, iidx_v, part_v, uval_v, ival_v, bias_v, sems):
        wid = lax.axis_index("s") * _NC + lax.axis_index("c")
        base = wid * bpw

        cu0 = pltpu.async_copy(uidx.at[pl.ds(base, bpw)], uidx_v, sems[0])
        ci0 = pltpu.async_copy(iidx.at[pl.ds(base, bpw)], iidx_v, sems[1])
        cp = pltpu.async_copy(part.at[pl.ds(base, bpw)], part_v, sems[2])
        cb = pltpu.async_copy(bias.at[pl.ds(0, 1)], bias_v.at[pl.ds(0, 1)],
                              sems[5])
        cu0.wait()
        cu = pltpu.async_copy(pu.at[uidx_v], uval_v, sems[3])
        ci0.wait()
        ci = pltpu.async_copy(pi.at[iidx_v], ival_v, sems[4])
        cp.wait()
        cu.wait()
        ci.wait()
        cb.wait()
        bvec = plsc.load_gather(bias_v, [jnp.zeros((_L,), jnp.int32)])
        for g in range(n_grp):
            sl = pl.ds(g * _L, _L)
            part_v[sl] = part_v[sl] + uval_v[sl] + ival_v[sl] + bvec
        pltpu.sync_copy(part_v, out.at[pl.ds(base, bpw)])

    return sc2


def kernel(user, item, sentence, feature, feature_len, user_table, item_table,
           feature_table, sent_table, fc_w, fc_b):
    b = user.shape[0]
    f_len = feature.shape[1]
    d_u = user_table.shape[1]
    d_i = item_table.shape[1]
    d_s = sent_table.shape[1]
    v_f = feature_table.shape[0]

    fcw = fc_w.astype(jnp.float32)
    w_u = fcw[:, :d_u]
    w_i = fcw[:, d_u:d_u + d_i]
    w_s = fcw[0, d_u + d_i:d_u + d_i + d_s]
    w_f = fcw[:, d_u + d_i + d_s:]

    pf = _project_feature_table(feature_table.T, w_f)
    pu, pi = _project_tables(user_table.T, item_table.T, w_u, w_i)

    fidx_t = feature.astype(jnp.int32).T
    flen_i = feature_len.astype(jnp.int32)

    sc1 = _make_sc1(b, f_len, d_s, v_f)
    part = sc1(sentence.astype(jnp.int32), fidx_t, flen_i,
               sent_table.astype(jnp.float32), pf, w_s)
    sc2 = _make_sc2(b)
    res = sc2(user.astype(jnp.int32), item.astype(jnp.int32), part, pu, pi,
              fc_b.astype(jnp.float32))
    return res.reshape(b, 1)


# speedup vs baseline: 11.4671x; 1.1473x over previous
import functools

import jax
import jax.numpy as jnp
from jax import lax
from jax.experimental import pallas as pl
from jax.experimental.pallas import tpu as pltpu
from jax.experimental.pallas import tpu_sc as plsc

_NC = 2
_NS = 16
_NW = _NC * _NS
_L = 16

_BLK = 16384


def _make_proj_body(d_u, d_i):
    def _proj_body(utt_ref, itt_ref, fcw_ref, pu_ref, pi_ref):
        wu = fcw_ref[:, :d_u]
        wi = fcw_ref[:, d_u:d_u + d_i]
        pu_ref[...] = jnp.dot(wu, utt_ref[...],
                              preferred_element_type=jnp.float32)[0]
        pi_ref[...] = jnp.dot(wi, itt_ref[...],
                              preferred_element_type=jnp.float32)[0]
    return _proj_body


def _project_tables(ut_t, it_t, fcw):
    d, v = ut_t.shape
    grid = (v + _BLK - 1) // _BLK
    return pl.pallas_call(
        _make_proj_body(ut_t.shape[0], it_t.shape[0]),
        grid=(grid,),
        in_specs=[
            pl.BlockSpec((d, _BLK), lambda j: (0, j)),
            pl.BlockSpec((d, _BLK), lambda j: (0, j)),
            pl.BlockSpec(fcw.shape, lambda j: (0, 0)),
        ],
        out_specs=[
            pl.BlockSpec((_BLK,), lambda j: (j,)),
            pl.BlockSpec((_BLK,), lambda j: (j,)),
        ],
        out_shape=[
            jax.ShapeDtypeStruct((v,), jnp.float32),
            jax.ShapeDtypeStruct((v,), jnp.float32),
        ],
    )(ut_t, it_t, fcw)


def _make_pf_body(woff, d_f):
    def _pf_body(ftt_ref, fcw_ref, pf_ref):
        wf = fcw_ref[:, woff:woff + d_f]
        pf_ref[...] = jnp.dot(wf, ftt_ref[...],
                              preferred_element_type=jnp.float32)[0]
    return _pf_body


def _project_feature_table(ft_t, fcw, woff):
    d, v_f = ft_t.shape
    return pl.pallas_call(
        _make_pf_body(woff, d),
        out_shape=jax.ShapeDtypeStruct((v_f,), jnp.float32),
    )(ft_t, fcw)


def _mesh():
    return plsc.VectorSubcoreMesh(
        core_axis_name="c", subcore_axis_name="s",
        num_cores=_NC, num_subcores=_NS)


def _make_sc1(b, f_len, d_s, v_f, ws_off):
    bpw = b // _NW
    n_grp = bpw // _L

    @functools.partial(
        pl.kernel,
        out_type=jax.ShapeDtypeStruct((b,), jnp.float32),
        mesh=_mesh(),
        compiler_params=pltpu.CompilerParams(
            needs_layout_passes=False, use_tc_tiling_on_sc=False),
        scratch_types=[
            pltpu.VMEM((bpw,), jnp.int32),
            pltpu.VMEM((f_len, bpw), jnp.int32),
            pltpu.VMEM((bpw,), jnp.int32),
            pltpu.VMEM((bpw, d_s), jnp.float32),
            pltpu.VMEM((v_f,), jnp.float32),
            pltpu.VMEM((d_s + _L,), jnp.float32),
            pltpu.VMEM((bpw,), jnp.float32),
            [pltpu.SemaphoreType.DMA] * 5,
            [pltpu.SemaphoreType.DMA] * 4,
        ],
    )
    def sc1(sidx, fidx_t, flen, st, pf, ws, out,
            sidx_v, fidx_v, flen_v, srows_v, pf_v, ws_v, out_v,
            sems, sems_s):
        wid = lax.axis_index("s") * _NC + lax.axis_index("c")
        base = wid * bpw

        c_stage = [
            pltpu.async_copy(sidx.at[pl.ds(base, bpw)], sidx_v, sems[0]),
            pltpu.async_copy(fidx_t.at[:, pl.ds(base, bpw)], fidx_v, sems[1]),
            pltpu.async_copy(flen.at[pl.ds(base, bpw)], flen_v, sems[2]),
            pltpu.async_copy(pf, pf_v, sems[3]),
            pltpu.async_copy(ws.at[pl.ds(ws_off, d_s)],
                             ws_v.at[pl.ds(0, d_s)], sems[4]),
        ]
        c_stage[0].wait()
        n_sp = len(sems_s)
        spr = bpw // n_sp
        cs = [
            pltpu.async_copy(st.at[sidx_v.at[pl.ds(k * spr, spr)]],
                             srows_v.at[pl.ds(k * spr, spr)], sems_s[k])
            for k in range(n_sp)]
        for c in c_stage[1:]:
            c.wait()

        for g in range(n_grp):
            facc = jnp.zeros((_L,), jnp.float32)
            lv = flen_v[pl.ds(g * _L, _L)].astype(jnp.float32)
            for f in range(f_len):
                idxv = fidx_v[f, pl.ds(g * _L, _L)]
                vals = plsc.load_gather(pf_v, [idxv])
                facc = facc + jnp.where(jnp.float32(f) < lv, vals, 0.0)
            out_v[pl.ds(g * _L, _L)] = facc / lv

        for c in cs:
            c.wait()

        lanes = lax.iota(jnp.int32, _L)
        ridx = [g * _L + lanes for g in range(n_grp)]
        accs = tuple(out_v[pl.ds(g * _L, _L)] for g in range(n_grp))

        def body(_, carry):
            col, *accs = carry
            wv = plsc.load_gather(ws_v, [col])
            accs = [
                a + plsc.load_gather(srows_v, [ridx[g], col]) * wv
                for g, a in enumerate(accs)]
            return (col + 1, *accs)

        carry = lax.fori_loop(
            0, d_s, body, (jnp.zeros((_L,), jnp.int32), *accs), unroll=4)
        for g in range(n_grp):
            out_v[pl.ds(g * _L, _L)] = carry[1 + g]

        pltpu.sync_copy(out_v, out.at[pl.ds(base, bpw)])

    return sc1


def _make_sc2(b):
    bpw = b // _NW
    n_grp = bpw // _L

    @functools.partial(
        pl.kernel,
        out_type=jax.ShapeDtypeStruct((b,), jnp.float32),
        mesh=_mesh(),
        compiler_params=pltpu.CompilerParams(
            needs_layout_passes=False, use_tc_tiling_on_sc=False),
        scratch_types=[
            pltpu.VMEM((bpw,), jnp.int32),
            pltpu.VMEM((bpw,), jnp.int32),
            pltpu.VMEM((bpw,), jnp.float32),
            pltpu.VMEM((bpw,), jnp.float32),
            pltpu.VMEM((bpw,), jnp.float32),
            pltpu.VMEM((_L,), jnp.float32),
            [pltpu.SemaphoreType.DMA] * 6,
        ],
    )
    def sc2(uidx, iidx, part, pu, pi, bias, out,
            uidx_v, iidx_v, part_v, uval_v, ival_v, bias_v, sems):
        wid = lax.axis_index("s") * _NC + lax.axis_index("c")
        base = wid * bpw

        cu0 = pltpu.async_copy(uidx.at[pl.ds(base, bpw)], uidx_v, sems[0])
        ci0 = pltpu.async_copy(iidx.at[pl.ds(base, bpw)], iidx_v, sems[1])
        cp = pltpu.async_copy(part.at[pl.ds(base, bpw)], part_v, sems[2])
        cb = pltpu.async_copy(bias.at[pl.ds(0, 1)], bias_v.at[pl.ds(0, 1)],
                              sems[5])
        cu0.wait()
        cu = pltpu.async_copy(pu.at[uidx_v], uval_v, sems[3])
        ci0.wait()
        ci = pltpu.async_copy(pi.at[iidx_v], ival_v, sems[4])
        cp.wait()
        cu.wait()
        ci.wait()
        cb.wait()
        bvec = plsc.load_gather(bias_v, [jnp.zeros((_L,), jnp.int32)])
        for g in range(n_grp):
            sl = pl.ds(g * _L, _L)
            part_v[sl] = part_v[sl] + uval_v[sl] + ival_v[sl] + bvec
        pltpu.sync_copy(part_v, out.at[pl.ds(base, bpw)])

    return sc2


def kernel(user, item, sentence, feature, feature_len, user_table, item_table,
           feature_table, sent_table, fc_w, fc_b):
    b = user.shape[0]
    f_len = feature.shape[1]
    d_u = user_table.shape[1]
    d_i = item_table.shape[1]
    d_s = sent_table.shape[1]
    v_f = feature_table.shape[0]

    fcw = fc_w.astype(jnp.float32)
    fcw_flat = fcw.reshape(-1)

    pf = _project_feature_table(feature_table.T, fcw, d_u + d_i + d_s)
    pu, pi = _project_tables(user_table.T, item_table.T, fcw)

    fidx_t = feature.astype(jnp.int32).T
    flen_i = feature_len.astype(jnp.int32)

    sc1 = _make_sc1(b, f_len, d_s, v_f, d_u + d_i)
    part = sc1(sentence.astype(jnp.int32), fidx_t, flen_i,
               sent_table.astype(jnp.float32), pf, fcw_flat)
    sc2 = _make_sc2(b)
    res = sc2(user.astype(jnp.int32), item.astype(jnp.int32), part, pu, pi,
              fc_b.astype(jnp.float32))
    return res.reshape(b, 1)


# speedup vs baseline: 11.5507x; 1.0073x over previous
import functools

import jax
import jax.numpy as jnp
from jax import lax
from jax.experimental import pallas as pl
from jax.experimental.pallas import tpu as pltpu
from jax.experimental.pallas import tpu_sc as plsc

_NC = 2
_NS = 16
_NW = _NC * _NS
_L = 16

_BLK = 25600


def _make_proj_body(d_u, d_i):
    def _proj_body(utt_ref, itt_ref, fcw_ref, pu_ref, pi_ref):
        wu = fcw_ref[:, :d_u]
        wi = fcw_ref[:, d_u:d_u + d_i]
        pu_ref[...] = jnp.dot(wu, utt_ref[...],
                              preferred_element_type=jnp.float32)[0]
        pi_ref[...] = jnp.dot(wi, itt_ref[...],
                              preferred_element_type=jnp.float32)[0]
    return _proj_body


def _project_tables(ut_t, it_t, fcw):
    d, v = ut_t.shape
    grid = (v + _BLK - 1) // _BLK
    return pl.pallas_call(
        _make_proj_body(ut_t.shape[0], it_t.shape[0]),
        grid=(grid,),
        in_specs=[
            pl.BlockSpec((d, _BLK), lambda j: (0, j)),
            pl.BlockSpec((d, _BLK), lambda j: (0, j)),
            pl.BlockSpec(fcw.shape, lambda j: (0, 0)),
        ],
        out_specs=[
            pl.BlockSpec((_BLK,), lambda j: (j,)),
            pl.BlockSpec((_BLK,), lambda j: (j,)),
        ],
        out_shape=[
            jax.ShapeDtypeStruct((v,), jnp.float32),
            jax.ShapeDtypeStruct((v,), jnp.float32),
        ],
    )(ut_t, it_t, fcw)


def _make_pf_body(woff, d_f):
    def _pf_body(ftt_ref, fcw_ref, pf_ref):
        wf = fcw_ref[:, woff:woff + d_f]
        pf_ref[...] = jnp.dot(wf, ftt_ref[...],
                              preferred_element_type=jnp.float32)[0]
    return _pf_body


def _project_feature_table(ft_t, fcw, woff):
    d, v_f = ft_t.shape
    return pl.pallas_call(
        _make_pf_body(woff, d),
        out_shape=jax.ShapeDtypeStruct((v_f,), jnp.float32),
    )(ft_t, fcw)


def _mesh():
    return plsc.VectorSubcoreMesh(
        core_axis_name="c", subcore_axis_name="s",
        num_cores=_NC, num_subcores=_NS)


def _make_sc1(b, f_len, d_s, v_f, ws_off):
    bpw = b // _NW
    n_grp = bpw // _L

    @functools.partial(
        pl.kernel,
        out_type=jax.ShapeDtypeStruct((b,), jnp.float32),
        mesh=_mesh(),
        compiler_params=pltpu.CompilerParams(
            needs_layout_passes=False, use_tc_tiling_on_sc=False),
        scratch_types=[
            pltpu.VMEM((bpw,), jnp.int32),
            pltpu.VMEM((f_len, bpw), jnp.int32),
            pltpu.VMEM((bpw,), jnp.int32),
            pltpu.VMEM((bpw, d_s), jnp.float32),
            pltpu.VMEM((v_f,), jnp.float32),
            pltpu.VMEM((d_s + _L,), jnp.float32),
            pltpu.VMEM((bpw,), jnp.float32),
            [pltpu.SemaphoreType.DMA] * 5,
            [pltpu.SemaphoreType.DMA] * 4,
        ],
    )
    def sc1(sidx, fidx_t, flen, st, pf, ws, out,
            sidx_v, fidx_v, flen_v, srows_v, pf_v, ws_v, out_v,
            sems, sems_s):
        wid = lax.axis_index("s") * _NC + lax.axis_index("c")
        base = wid * bpw

        c_stage = [
            pltpu.async_copy(sidx.at[pl.ds(base, bpw)], sidx_v, sems[0]),
            pltpu.async_copy(fidx_t.at[:, pl.ds(base, bpw)], fidx_v, sems[1]),
            pltpu.async_copy(flen.at[pl.ds(base, bpw)], flen_v, sems[2]),
            pltpu.async_copy(pf, pf_v, sems[3]),
            pltpu.async_copy(ws.at[pl.ds(ws_off, d_s)],
                             ws_v.at[pl.ds(0, d_s)], sems[4]),
        ]
        c_stage[0].wait()
        n_sp = len(sems_s)
        spr = bpw // n_sp
        cs = [
            pltpu.async_copy(st.at[sidx_v.at[pl.ds(k * spr, spr)]],
                             srows_v.at[pl.ds(k * spr, spr)], sems_s[k])
            for k in range(n_sp)]
        for c in c_stage[1:]:
            c.wait()

        for g in range(n_grp):
            facc = jnp.zeros((_L,), jnp.float32)
            lv = flen_v[pl.ds(g * _L, _L)].astype(jnp.float32)
            for f in range(f_len):
                idxv = fidx_v[f, pl.ds(g * _L, _L)]
                vals = plsc.load_gather(pf_v, [idxv])
                facc = facc + jnp.where(jnp.float32(f) < lv, vals, 0.0)
            out_v[pl.ds(g * _L, _L)] = facc / lv

        for c in cs:
            c.wait()

        lanes = lax.iota(jnp.int32, _L)
        ridx = [g * _L + lanes for g in range(n_grp)]
        accs = tuple(out_v[pl.ds(g * _L, _L)] for g in range(n_grp))

        def body(_, carry):
            col, *accs = carry
            wv = plsc.load_gather(ws_v, [col])
            accs = [
                a + plsc.load_gather(srows_v, [ridx[g], col]) * wv
                for g, a in enumerate(accs)]
            return (col + 1, *accs)

        carry = lax.fori_loop(
            0, d_s, body, (jnp.zeros((_L,), jnp.int32), *accs), unroll=4)
        for g in range(n_grp):
            out_v[pl.ds(g * _L, _L)] = carry[1 + g]

        pltpu.sync_copy(out_v, out.at[pl.ds(base, bpw)])

    return sc1


def _make_sc2(b):
    bpw = b // _NW
    n_grp = bpw // _L

    @functools.partial(
        pl.kernel,
        out_type=jax.ShapeDtypeStruct((b,), jnp.float32),
        mesh=_mesh(),
        compiler_params=pltpu.CompilerParams(
            needs_layout_passes=False, use_tc_tiling_on_sc=False),
        scratch_types=[
            pltpu.VMEM((bpw,), jnp.int32),
            pltpu.VMEM((bpw,), jnp.int32),
            pltpu.VMEM((bpw,), jnp.float32),
            pltpu.VMEM((bpw,), jnp.float32),
            pltpu.VMEM((bpw,), jnp.float32),
            pltpu.VMEM((_L,), jnp.float32),
            [pltpu.SemaphoreType.DMA] * 6,
        ],
    )
    def sc2(uidx, iidx, part, pu, pi, bias, out,
            uidx_v, iidx_v, part_v, uval_v, ival_v, bias_v, sems):
        wid = lax.axis_index("s") * _NC + lax.axis_index("c")
        base = wid * bpw

        cu0 = pltpu.async_copy(uidx.at[pl.ds(base, bpw)], uidx_v, sems[0])
        ci0 = pltpu.async_copy(iidx.at[pl.ds(base, bpw)], iidx_v, sems[1])
        cp = pltpu.async_copy(part.at[pl.ds(base, bpw)], part_v, sems[2])
        cb = pltpu.async_copy(bias.at[pl.ds(0, 1)], bias_v.at[pl.ds(0, 1)],
                              sems[5])
        cu0.wait()
        cu = pltpu.async_copy(pu.at[uidx_v], uval_v, sems[3])
        ci0.wait()
        ci = pltpu.async_copy(pi.at[iidx_v], ival_v, sems[4])
        cp.wait()
        cu.wait()
        ci.wait()
        cb.wait()
        bvec = plsc.load_gather(bias_v, [jnp.zeros((_L,), jnp.int32)])
        for g in range(n_grp):
            sl = pl.ds(g * _L, _L)
            part_v[sl] = part_v[sl] + uval_v[sl] + ival_v[sl] + bvec
        pltpu.sync_copy(part_v, out.at[pl.ds(base, bpw)])

    return sc2


def kernel(user, item, sentence, feature, feature_len, user_table, item_table,
           feature_table, sent_table, fc_w, fc_b):
    b = user.shape[0]
    f_len = feature.shape[1]
    d_u = user_table.shape[1]
    d_i = item_table.shape[1]
    d_s = sent_table.shape[1]
    v_f = feature_table.shape[0]

    fcw = fc_w.astype(jnp.float32)
    fcw_flat = fcw.reshape(-1)

    pf = _project_feature_table(feature_table.T, fcw, d_u + d_i + d_s)
    pu, pi = _project_tables(user_table.T, item_table.T, fcw)

    fidx_t = feature.astype(jnp.int32).T
    flen_i = feature_len.astype(jnp.int32)

    sc1 = _make_sc1(b, f_len, d_s, v_f, d_u + d_i)
    part = sc1(sentence.astype(jnp.int32), fidx_t, flen_i,
               sent_table.astype(jnp.float32), pf, fcw_flat)
    sc2 = _make_sc2(b)
    res = sc2(user.astype(jnp.int32), item.astype(jnp.int32), part, pu, pi,
              fc_b.astype(jnp.float32))
    return res.reshape(b, 1)
